# baseline (speedup 1.0000x reference)
import functools

import jax
import jax.numpy as jnp
from jax import lax
from jax.experimental import pallas as pl
from jax.experimental.pallas import tpu as pltpu
from jax.experimental.pallas import tpu_sc as plsc

_NC = 2
_NS = 16
_NW = _NC * _NS
_CHUNK = 128
_PACK = 8


def _sc_gather(user_tp, item_tp, user_pidx, item_pidx):
    B = user_pidx.shape[0]
    W = user_tp.shape[1]
    bpw = B // _NW
    cpw = bpw // _CHUNK

    mesh = plsc.VectorSubcoreMesh(core_axis_name="c", subcore_axis_name="s")

    @functools.partial(
        pl.kernel,
        mesh=mesh,
        out_type=(jax.ShapeDtypeStruct((B, W), jnp.float32),
                  jax.ShapeDtypeStruct((B, W), jnp.float32)),
        scratch_types=[
            pltpu.VMEM((cpw, _CHUNK), jnp.int32),
            pltpu.VMEM((cpw, _CHUNK), jnp.int32),
            pltpu.VMEM((2, _CHUNK, 128), jnp.float32),
            pltpu.VMEM((2, _CHUNK, 128), jnp.float32),
            pltpu.SemaphoreType.DMA,
            pltpu.SemaphoreType.DMA,
            pltpu.SemaphoreType.DMA,
        ],
    )
    def sc_k(ut_hbm, it_hbm, ui_hbm, ii_hbm, uo_hbm, io_hbm,
             uidx_v, iidx_v, urows_v, irows_v, gsem_u, gsem_i, osem):
        wid = lax.axis_index("s") * _NC + lax.axis_index("c")
        pltpu.sync_copy(ui_hbm.at[pl.ds(wid * cpw, cpw)], uidx_v)
        pltpu.sync_copy(ii_hbm.at[pl.ds(wid * cpw, cpw)], iidx_v)

        def fire(c):
            b = c % 2
            return (pltpu.async_copy(ut_hbm.at[uidx_v.at[c]],
                                     urows_v.at[b], gsem_u),
                    pltpu.async_copy(it_hbm.at[iidx_v.at[c]],
                                     irows_v.at[b], gsem_i))

        g = [None] * cpw
        o = [None] * cpw
        g[0] = fire(0)
        for c in range(cpw):
            for h in g[c]:
                h.wait()
            b = c % 2
            row0 = wid * bpw + c * _CHUNK
            o[c] = (pltpu.async_copy(urows_v.at[b],
                                     uo_hbm.at[pl.ds(row0, _CHUNK)], osem),
                    pltpu.async_copy(irows_v.at[b],
                                     io_hbm.at[pl.ds(row0, _CHUNK)], osem))
            if c + 1 < cpw:
                if c >= 1:
                    for h in o[c - 1]:
                        h.wait()
                g[c + 1] = fire(c + 1)
        for c in (cpw - 2, cpw - 1):
            if c >= 0 and o[c] is not None:
                for h in o[c]:
                    h.wait()

    ui2 = user_pidx.reshape(B // _CHUNK, _CHUNK)
    ii2 = item_pidx.reshape(B // _CHUNK, _CHUNK)
    return sc_k(user_tp, item_tp, ui2, ii2)


def _select_sub(x, sel, D):
    acc = jnp.where(sel == 0, x[:, 0:D], 0.0)
    for p in range(1, _PACK):
        acc = acc + jnp.where(sel == p, x[:, p * D:(p + 1) * D], 0.0)
    return acc


def _mlp_body(xu_ref, xi_ref, su_ref, si_ref, w1u_ref, w1i_ref, b1_ref,
              w2_ref, b2_ref, w3_ref, b3_ref, wf_ref, bf_ref, o_ref):
    D = w1u_ref.shape[0]
    u = _select_sub(xu_ref[...], su_ref[...], D)
    it = _select_sub(xi_ref[...], si_ref[...], D)
    hp = jax.lax.Precision.HIGHEST
    h = jnp.dot(u, w1u_ref[...], precision=hp)
    h += jnp.dot(it, w1i_ref[...], precision=hp)
    h = jnp.maximum(h + b1_ref[...], 0.0)
    h = jnp.maximum(jnp.dot(h, w2_ref[...], precision=hp) + b2_ref[...], 0.0)
    h = jnp.maximum(jnp.dot(h, w3_ref[...], precision=hp) + b3_ref[...], 0.0)
    logits = jnp.dot(h, wf_ref[...], precision=hp) + bf_ref[...]
    o_ref[...] = jax.nn.sigmoid(logits)


def _tc_mlp(xu, xi, su, si, W1, b1, W2, b2, W3, b3, Wf, bf):
    B = xu.shape[0]
    M = W1.shape[0] // 2
    blk = 2048
    w1u, w1i = W1[:M], W1[M:]
    b1r, b2r, b3r = b1.reshape(1, -1), b2.reshape(1, -1), b3.reshape(1, -1)
    bfr = bf.reshape(1, 1)

    full = lambda shape: pl.BlockSpec(shape, lambda b: (0, 0))
    return pl.pallas_call(
        _mlp_body,
        grid=(B // blk,),
        in_specs=[
            pl.BlockSpec((blk, 128), lambda b: (b, 0)),
            pl.BlockSpec((blk, 128), lambda b: (b, 0)),
            pl.BlockSpec((blk, 1), lambda b: (b, 0)),
            pl.BlockSpec((blk, 1), lambda b: (b, 0)),
            full(w1u.shape), full(w1i.shape), full(b1r.shape),
            full(W2.shape), full(b2r.shape),
            full(W3.shape), full(b3r.shape),
            full(Wf.shape), full(bfr.shape),
        ],
        out_specs=pl.BlockSpec((blk, 1), lambda b: (b, 0)),
        out_shape=jax.ShapeDtypeStruct((B, 1), jnp.float32),
        compiler_params=pltpu.CompilerParams(
            dimension_semantics=("parallel",)),
    )(xu, xi, su, si, w1u, w1i, b1r, W2, b2r, W3, b3r, Wf, bfr)


def kernel(user_input, item_input, user_table, item_table,
           W1, b1, W2, b2, W3, b3, Wf, bf):
    V, D = user_table.shape
    W = _PACK * D
    utp = user_table.reshape(V * D // W, W)
    itp = item_table.reshape(V * D // W, W)
    u_pidx = lax.shift_right_logical(user_input, 3)
    i_pidx = lax.shift_right_logical(item_input, 3)
    xu, xi = _sc_gather(utp, itp, u_pidx, i_pidx)
    su = (user_input & 7).astype(jnp.float32).reshape(-1, 1)
    si = (item_input & 7).astype(jnp.float32).reshape(-1, 1)
    return _tc_mlp(xu, xi, su, si, W1, b1, W2, b2, W3, b3, Wf, bf)
